# baseline (speedup 1.0000x reference)
import functools

import jax
import jax.numpy as jnp
from jax import lax
from jax.experimental import pallas as pl
from jax.experimental.pallas import tpu as pltpu
from jax.experimental.pallas import tpu_sc as plsc

N = 1024
E = 32768
H = 2

NS = 16
EPW = E // NS
CHUNK = 128
NCH = EPW // CHUNK
SLICE = (N * N) // NS
ZBUF = 8192


def _f32dot(a, b):
    ah = a.astype(jnp.bfloat16)
    al = (a - ah.astype(jnp.float32)).astype(jnp.bfloat16)
    bh = b.astype(jnp.bfloat16)
    bl = (b - bh.astype(jnp.float32)).astype(jnp.bfloat16)
    d = functools.partial(jnp.dot, preferred_element_type=jnp.float32)
    return ((d(al, bl) + d(al, bh)) + (d(ah, bl) + d(ah, bh)))


def _bdot(a, b):
    return jnp.dot(a.astype(jnp.bfloat16), b.astype(jnp.bfloat16),
                   preferred_element_type=jnp.float32)


def _attn_logits(xp, a_flat_ref, o1):
    t = xp * a_flat_ref[...]
    return jnp.concatenate(
        [jnp.sum(t[:, h * o1:(h + 1) * o1], axis=1, keepdims=True) for h in range(H)],
        axis=1)


def _normed_agg(a_ref, xp, o1):
    cols = []
    for h in range(H):
        A = a_ref[h]
        U = _f32dot(A, xp[:, h * o1:(h + 1) * o1])
        den = jnp.sum(A, axis=1, keepdims=True)
        cols.append(U / (den + 1e-16))
    return jnp.concatenate(cols, axis=1)


def _pre_body(o1, x_ref, w_ref, asrc_ref, adst_ref, xp_ref, al_ref):
    xp = _bdot(x_ref[...], w_ref[...])
    xp_ref[...] = xp
    al_ref[...] = jnp.concatenate(
        [_attn_logits(xp, asrc_ref, o1), _attn_logits(xp, adst_ref, o1)], axis=1)


def _pre(x, W, a_src_flat, a_dst_flat):
    dout = W.shape[1]
    return pl.pallas_call(
        functools.partial(_pre_body, dout // H),
        out_shape=[
            jax.ShapeDtypeStruct((N, dout), jnp.float32),
            jax.ShapeDtypeStruct((N, 2 * H), jnp.float32),
        ],
    )(x, W, a_src_flat, a_dst_flat)


def _mid_body(o1, o2, a_ref, xp_ref, b_ref, w2_ref, asrc_ref, adst_ref,
              xp2_ref, al_ref):
    h1 = jnp.maximum(_normed_agg(a_ref, xp_ref[...], o1) + b_ref[...], 0.0)
    xp2 = _bdot(h1, w2_ref[...])
    xp2_ref[...] = xp2
    al_ref[...] = jnp.concatenate(
        [_attn_logits(xp2, asrc_ref, o2), _attn_logits(xp2, adst_ref, o2)], axis=1)


def _mid(A1, xp1, b1_row, W2, a_src_flat, a_dst_flat):
    o1 = xp1.shape[1] // H
    dout = W2.shape[1]
    return pl.pallas_call(
        functools.partial(_mid_body, o1, dout // H),
        out_shape=[
            jax.ShapeDtypeStruct((N, dout), jnp.float32),
            jax.ShapeDtypeStruct((N, 2 * H), jnp.float32),
        ],
    )(A1, xp1, b1_row, W2, a_src_flat, a_dst_flat)


BI = 32
NBLK = N // BI

def _fin_body(o2, a_ref, xp_ref, b_ref, w_ref, lb_ref, out_ref, h2_s, pb_s):
    bi = pl.program_id(0)

    @pl.when(bi == 0)
    def _():
        h2_s[...] = _normed_agg(a_ref, xp_ref[...], o2) + b_ref[...]

    @pl.when(bi < NBLK)
    def _():
        hi = h2_s[pl.ds(bi * BI, BI), :]
        pb_s[bi % 2] = (hi[:, None, :] + h2_s[...][None, :, :]).astype(jnp.bfloat16)

    @pl.when(bi > 0)
    def _():
        wb = w_ref[...].astype(jnp.bfloat16)
        ob = lax.dot_general(wb, pb_s[(bi - 1) % 2], (((0,), (2,)), ((), ())),
                             preferred_element_type=jnp.float32)
        out_ref[...] = ob[0] + lb_ref[0, 0]


def _fin(A2, xp2, b2_row, lin_W, lin_b_2d):
    o2 = xp2.shape[1] // H
    return pl.pallas_call(
        functools.partial(_fin_body, o2),
        grid=(NBLK + 1,),
        in_specs=[
            pl.BlockSpec((H, N, N), lambda i: (0, 0, 0)),
            pl.BlockSpec((N, H * o2), lambda i: (0, 0)),
            pl.BlockSpec((1, H * o2), lambda i: (0, 0)),
            pl.BlockSpec((H * o2, 1), lambda i: (0, 0)),
            pl.BlockSpec((1, 1), lambda i: (0, 0)),
        ],
        out_specs=pl.BlockSpec((BI, N), lambda i: (jnp.maximum(i - 1, 0), 0)),
        out_shape=jax.ShapeDtypeStruct((N, N), jnp.float32),
        scratch_shapes=[
            pltpu.VMEM((N, H * o2), jnp.float32),
            pltpu.VMEM((2, BI, N, H * o2), jnp.bfloat16),
        ],
    )(A2, xp2, b2_row, lin_W, lin_b_2d)


def _abuild_body(src_hbm, dst_hbm, al_hbm, out_hbm,
                 src_v, dst_v, al_v, vals_v, idx_v, zero_v,
                 a_sh, sem_a, sem_z):
    c = lax.axis_index("c")
    s = lax.axis_index("s")

    ebase = s * EPW
    stage = [
        pltpu.async_copy(src_hbm.at[pl.ds(ebase, EPW)], src_v, sem_a),
        pltpu.async_copy(dst_hbm.at[pl.ds(ebase, EPW)], dst_v, sem_a),
        pltpu.async_copy(al_hbm, al_v, sem_a),
    ]

    for i in range(ZBUF // 16):
        zero_v[pl.ds(i * 16, 16)] = jnp.zeros((16,), jnp.float32)
    abase = s * SLICE
    zcopies = [
        pltpu.async_copy(zero_v, a_sh.at[pl.ds(abase + k * ZBUF, ZBUF)], sem_z)
        for k in range(SLICE // ZBUF)
    ]

    for cp in stage:
        cp.wait()

    for r in range(NCH):
        for q in range(CHUNK // 16):
            off = r * CHUNK + q * 16
            s16 = src_v[pl.ds(off, 16)]
            d16 = dst_v[pl.ds(off, 16)]
            av = plsc.load_gather(al_v, [s16 * (2 * H) + c])
            bv = plsc.load_gather(al_v, [d16 * (2 * H) + H + c])
            e = av + bv
            e = jnp.where(e >= 0.0, e, e * 0.2)
            vals_v[r, pl.ds(q * 16, 16)] = jnp.exp(e)
            idx_v[r, pl.ds(q * 16, 16)] = d16 * N + s16

    for cp in zcopies:
        cp.wait()
    plsc.subcore_barrier()

    adds = [
        pltpu.async_copy(vals_v.at[r], a_sh.at[idx_v.at[r]], sem_a, add=True)
        for r in range(NCH)
    ]
    for cp in adds:
        cp.wait()

    plsc.subcore_barrier()
    rows = N // NS
    outs = [
        pltpu.async_copy(a_sh.at[pl.ds(abase + r * N, N)],
                         out_hbm.at[c, s * rows + r], sem_a)
        for r in range(rows)
    ]
    for cp in outs:
        cp.wait()


def _abuild(src, dst, al_flat):
    mesh = plsc.VectorSubcoreMesh(core_axis_name="c", subcore_axis_name="s")
    k = pl.kernel(
        _abuild_body,
        out_type=jax.ShapeDtypeStruct((H, N, N), jnp.float32),
        mesh=mesh,
        compiler_params=pltpu.CompilerParams(needs_layout_passes=False),
        scratch_types=[
            pltpu.VMEM((EPW,), jnp.int32),
            pltpu.VMEM((EPW,), jnp.int32),
            pltpu.VMEM((N * 2 * H,), jnp.float32),
            pltpu.VMEM((NCH, CHUNK), jnp.float32),
            pltpu.VMEM((NCH, CHUNK), jnp.int32),
            pltpu.VMEM((ZBUF,), jnp.float32),
            pltpu.VMEM_SHARED((N * N,), jnp.float32),
            pltpu.SemaphoreType.DMA,
            pltpu.SemaphoreType.DMA,
        ],
    )
    return k(src, dst, al_flat)


@jax.jit
def kernel(x_node, edge_index_node_to_node, W1, a_src1, a_dst1, b1,
           W2, a_src2, a_dst2, b2, lin_W, lin_b):
    src = edge_index_node_to_node[0].astype(jnp.int32)
    dst = edge_index_node_to_node[1].astype(jnp.int32)

    xp1, al1 = _pre(x_node, W1, a_src1.reshape(1, -1), a_dst1.reshape(1, -1))
    A1 = _abuild(src, dst, al1.reshape(-1))
    xp2, al2 = _mid(A1, xp1, b1.reshape(1, -1), W2,
                    a_src2.reshape(1, -1), a_dst2.reshape(1, -1))
    A2 = _abuild(src, dst, al2.reshape(-1))
    out = _fin(A2, xp2, b2.reshape(1, -1), lin_W, lin_b.reshape(1, 1))
    return out.reshape(N * N, 1)


# speedup vs baseline: 1.3127x; 1.3127x over previous
import functools

import jax
import jax.numpy as jnp
from jax import lax
from jax.experimental import pallas as pl
from jax.experimental.pallas import tpu as pltpu
from jax.experimental.pallas import tpu_sc as plsc

N = 1024
E = 32768
H = 2

NS = 16
EPW = E // NS
CHUNK = 128
NCH = EPW // CHUNK
SLICE = (N * N) // NS
ZBUF = 8192


def _f32dot(a, b):
    ah = a.astype(jnp.bfloat16)
    al = (a - ah.astype(jnp.float32)).astype(jnp.bfloat16)
    bh = b.astype(jnp.bfloat16)
    bl = (b - bh.astype(jnp.float32)).astype(jnp.bfloat16)
    d = functools.partial(jnp.dot, preferred_element_type=jnp.float32)
    return ((d(al, bl) + d(al, bh)) + (d(ah, bl) + d(ah, bh)))


def _bdot(a, b):
    return jnp.dot(a.astype(jnp.bfloat16), b.astype(jnp.bfloat16),
                   preferred_element_type=jnp.float32)


def _attn_logits(xp, a_flat_ref, o1):
    t = xp * a_flat_ref[...]
    return jnp.concatenate(
        [jnp.sum(t[:, h * o1:(h + 1) * o1], axis=1, keepdims=True) for h in range(H)],
        axis=1)


def _normed_agg(a_ref, xp, o1):
    cols = []
    for h in range(H):
        A = a_ref[h]
        U = _f32dot(A, xp[:, h * o1:(h + 1) * o1])
        den = jnp.sum(A, axis=1, keepdims=True)
        cols.append(U / (den + 1e-16))
    return jnp.concatenate(cols, axis=1)


def _pre_body(o1, x_ref, w_ref, asrc_ref, adst_ref, xp_ref, al_ref):
    xp = _bdot(x_ref[...], w_ref[...])
    xp_ref[...] = xp
    al_ref[...] = jnp.concatenate(
        [_attn_logits(xp, asrc_ref, o1), _attn_logits(xp, adst_ref, o1)], axis=1)


def _pre(x, W, a_src_flat, a_dst_flat):
    dout = W.shape[1]
    return pl.pallas_call(
        functools.partial(_pre_body, dout // H),
        out_shape=[
            jax.ShapeDtypeStruct((N, dout), jnp.float32),
            jax.ShapeDtypeStruct((N, 2 * H), jnp.float32),
        ],
    )(x, W, a_src_flat, a_dst_flat)


def _mid_body(o1, o2, a_ref, xp_ref, b_ref, w2_ref, asrc_ref, adst_ref,
              xp2_ref, al_ref):
    h1 = jnp.maximum(_normed_agg(a_ref, xp_ref[...], o1) + b_ref[...], 0.0)
    xp2 = _bdot(h1, w2_ref[...])
    xp2_ref[...] = xp2
    al_ref[...] = jnp.concatenate(
        [_attn_logits(xp2, asrc_ref, o2), _attn_logits(xp2, adst_ref, o2)], axis=1)


def _mid(A1, xp1, b1_row, W2, a_src_flat, a_dst_flat):
    o1 = xp1.shape[1] // H
    dout = W2.shape[1]
    return pl.pallas_call(
        functools.partial(_mid_body, o1, dout // H),
        out_shape=[
            jax.ShapeDtypeStruct((N, dout), jnp.float32),
            jax.ShapeDtypeStruct((N, 2 * H), jnp.float32),
        ],
    )(A1, xp1, b1_row, W2, a_src_flat, a_dst_flat)


BI = 64

def _fin_body(o2, a_ref, xp_ref, b_ref, w_ref, lb_ref, out_ref, h2_s):
    bi = pl.program_id(0)

    @pl.when(bi == 0)
    def _():
        h2_s[...] = _normed_agg(a_ref, xp_ref[...], o2) + b_ref[...]

    hi = h2_s[pl.ds(bi * BI, BI), :]
    pb = (hi[:, None, :] + h2_s[...][None, :, :]).astype(jnp.bfloat16)
    wb = w_ref[...].astype(jnp.bfloat16)
    ob = lax.dot_general(wb, pb, (((0,), (2,)), ((), ())),
                         preferred_element_type=jnp.float32)
    out_ref[...] = (ob[0] + lb_ref[0, 0]).reshape(BI * 8, 128)


def _fin(A2, xp2, b2_row, lin_W, lin_b_2d):
    o2 = xp2.shape[1] // H
    return pl.pallas_call(
        functools.partial(_fin_body, o2),
        grid=(N // BI,),
        in_specs=[
            pl.BlockSpec((H, N, N), lambda i: (0, 0, 0)),
            pl.BlockSpec((N, H * o2), lambda i: (0, 0)),
            pl.BlockSpec((1, H * o2), lambda i: (0, 0)),
            pl.BlockSpec((H * o2, 1), lambda i: (0, 0)),
            pl.BlockSpec((1, 1), lambda i: (0, 0)),
        ],
        out_specs=pl.BlockSpec((BI * 8, 128), lambda i: (i, 0)),
        out_shape=jax.ShapeDtypeStruct((N * N // 128, 128), jnp.float32),
        scratch_shapes=[pltpu.VMEM((N, H * o2), jnp.float32)],
    )(A2, xp2, b2_row, lin_W, lin_b_2d)


def _abuild_body(src_hbm, dst_hbm, al_hbm, out_hbm,
                 src_v, dst_v, al_v, vals_v, idx_v, zero_v,
                 a_sh, sem_a, sem_z):
    c = lax.axis_index("c")
    s = lax.axis_index("s")

    ebase = s * EPW
    stage = [
        pltpu.async_copy(src_hbm.at[pl.ds(ebase, EPW)], src_v, sem_a),
        pltpu.async_copy(dst_hbm.at[pl.ds(ebase, EPW)], dst_v, sem_a),
        pltpu.async_copy(al_hbm, al_v, sem_a),
    ]

    for i in range(ZBUF // 16):
        zero_v[pl.ds(i * 16, 16)] = jnp.zeros((16,), jnp.float32)
    abase = s * SLICE
    zcopies = [
        pltpu.async_copy(zero_v, a_sh.at[pl.ds(abase + k * ZBUF, ZBUF)], sem_z)
        for k in range(SLICE // ZBUF)
    ]

    for cp in stage:
        cp.wait()

    for r in range(NCH):
        for q in range(CHUNK // 16):
            off = r * CHUNK + q * 16
            s16 = src_v[pl.ds(off, 16)]
            d16 = dst_v[pl.ds(off, 16)]
            av = plsc.load_gather(al_v, [s16 * (2 * H) + c])
            bv = plsc.load_gather(al_v, [d16 * (2 * H) + H + c])
            e = av + bv
            e = jnp.where(e >= 0.0, e, e * 0.2)
            vals_v[r, pl.ds(q * 16, 16)] = jnp.exp(e)
            idx_v[r, pl.ds(q * 16, 16)] = d16 * N + s16

    for cp in zcopies:
        cp.wait()
    plsc.subcore_barrier()

    adds = [
        pltpu.async_copy(vals_v.at[r], a_sh.at[idx_v.at[r]], sem_a, add=True)
        for r in range(NCH)
    ]
    for cp in adds:
        cp.wait()

    plsc.subcore_barrier()
    rows = N // NS
    outs = [
        pltpu.async_copy(a_sh.at[pl.ds(abase + r * N, N)],
                         out_hbm.at[c, s * rows + r], sem_a)
        for r in range(rows)
    ]
    for cp in outs:
        cp.wait()


def _abuild(src, dst, al_flat):
    mesh = plsc.VectorSubcoreMesh(core_axis_name="c", subcore_axis_name="s")
    k = pl.kernel(
        _abuild_body,
        out_type=jax.ShapeDtypeStruct((H, N, N), jnp.float32),
        mesh=mesh,
        compiler_params=pltpu.CompilerParams(needs_layout_passes=False),
        scratch_types=[
            pltpu.VMEM((EPW,), jnp.int32),
            pltpu.VMEM((EPW,), jnp.int32),
            pltpu.VMEM((N * 2 * H,), jnp.float32),
            pltpu.VMEM((NCH, CHUNK), jnp.float32),
            pltpu.VMEM((NCH, CHUNK), jnp.int32),
            pltpu.VMEM((ZBUF,), jnp.float32),
            pltpu.VMEM_SHARED((N * N,), jnp.float32),
            pltpu.SemaphoreType.DMA,
            pltpu.SemaphoreType.DMA,
        ],
    )
    return k(src, dst, al_flat)


@jax.jit
def kernel(x_node, edge_index_node_to_node, W1, a_src1, a_dst1, b1,
           W2, a_src2, a_dst2, b2, lin_W, lin_b):
    src = edge_index_node_to_node[0].astype(jnp.int32)
    dst = edge_index_node_to_node[1].astype(jnp.int32)

    xp1, al1 = _pre(x_node, W1, a_src1.reshape(1, -1), a_dst1.reshape(1, -1))
    A1 = _abuild(src, dst, al1.reshape(-1))
    xp2, al2 = _mid(A1, xp1, b1.reshape(1, -1), W2,
                    a_src2.reshape(1, -1), a_dst2.reshape(1, -1))
    A2 = _abuild(src, dst, al2.reshape(-1))
    out = _fin(A2, xp2, b2.reshape(1, -1), lin_W, lin_b.reshape(1, 1))
    return out.reshape(N * N, 1)
